# baseline (speedup 1.0000x reference)
import jax
import jax.numpy as jnp
from jax import lax
from jax.experimental import pallas as pl
from jax.experimental.pallas import tpu as pltpu
from jax.experimental.pallas import tpu_sc as plsc

_N = 16384
_C = 1000
_NW = 32
_RPW = _N // _NW
_CH = 32
_NCH = _RPW // _CH
_NBUF = 3


def _sc_body(x_hbm, o_hbm, buf, idx_v, sems):
    cid = lax.axis_index("c")
    sid = lax.axis_index("s")
    wid = sid * 2 + cid
    base = wid * _RPW

    z = jnp.zeros((16,), jnp.float32)

    def zero_row(r, _):
        for b in range(_NBUF):
            for c in range(_C // 16):
                buf[b, r, pl.ds(c * 16, 16)] = z
            buf[b, r, pl.ds(_C - 16, 16)] = z
        return _

    lax.fori_loop(0, _CH, zero_row, None)

    pltpu.sync_copy(x_hbm.at[pl.ds(base, _RPW)], idx_v)

    ones = jnp.full((16,), 1.0, jnp.float32)
    lane = lax.iota(jnp.int32, 16)

    def scatter(k, val):
        s = k % _NBUF
        for g in range(_CH // 16):
            rvec = g * 16 + lane
            xv = idx_v[pl.ds(k * _CH + g * 16, 16)]
            plsc.store_scatter(buf.at[s], [rvec, xv], val)

    def copy(k):
        return pltpu.make_async_copy(
            buf.at[k % _NBUF],
            o_hbm.at[pl.ds(base + k * _CH, _CH), :],
            sems.at[k % _NBUF],
        )

    for k in range(_NCH):
        if k >= _NBUF:
            copy(k - _NBUF).wait()
            scatter(k - _NBUF, z)
        scatter(k, ones)
        copy(k).start()
    for k in range(_NCH - _NBUF, _NCH):
        copy(k).wait()


_mesh = plsc.VectorSubcoreMesh(core_axis_name="c", subcore_axis_name="s")

_sc_call = pl.kernel(
    _sc_body,
    out_type=jax.ShapeDtypeStruct((_N, _C), jnp.float32),
    mesh=_mesh,
    scratch_types=[
        pltpu.VMEM((_NBUF, _CH, _C), jnp.float32),
        pltpu.VMEM((_RPW,), jnp.int32),
        pltpu.SemaphoreType.DMA((_NBUF,)),
    ],
    compiler_params=pltpu.CompilerParams(needs_layout_passes=False),
)


def kernel(x):
    return _sc_call(x.astype(jnp.int32))


# speedup vs baseline: 1.0140x; 1.0140x over previous
import jax
import jax.numpy as jnp
from jax import lax
from jax.experimental import pallas as pl
from jax.experimental.pallas import tpu as pltpu
from jax.experimental.pallas import tpu_sc as plsc

_N = 16384
_C = 1000
_NW = 32
_RPW = _N // _NW
_CH = 32
_NCH = _RPW // _CH
_NBUF = 2


def _sc_body(x_hbm, o_hbm, buf, idx_v, sems):
    cid = lax.axis_index("c")
    sid = lax.axis_index("s")
    wid = sid * 2 + cid
    base = wid * _RPW

    z = jnp.zeros((16,), jnp.float32)

    def zero_row(r, _):
        for b in range(_NBUF):
            for c in range(_C // 16):
                buf[b, r, pl.ds(c * 16, 16)] = z
            buf[b, r, pl.ds(_C - 16, 16)] = z
        return _

    lax.fori_loop(0, _CH, zero_row, None)

    pltpu.sync_copy(x_hbm.at[pl.ds(base, _RPW)], idx_v)

    ones = jnp.full((16,), 1.0, jnp.float32)
    lane = lax.iota(jnp.int32, 16)

    def scatter(k, val):
        s = k % _NBUF
        for g in range(_CH // 16):
            rvec = g * 16 + lane
            xv = idx_v[pl.ds(k * _CH + g * 16, 16)]
            plsc.store_scatter(buf.at[s], [rvec, xv], val)

    def copy(k):
        return pltpu.make_async_copy(
            buf.at[k % _NBUF],
            o_hbm.at[pl.ds(base + k * _CH, _CH), :],
            sems.at[k % _NBUF],
        )

    for k in range(_NCH):
        if k >= _NBUF:
            copy(k - _NBUF).wait()
            scatter(k - _NBUF, z)
        scatter(k, ones)
        copy(k).start()
    for k in range(_NCH - _NBUF, _NCH):
        copy(k).wait()


_mesh = plsc.VectorSubcoreMesh(core_axis_name="c", subcore_axis_name="s")

_sc_call = pl.kernel(
    _sc_body,
    out_type=jax.ShapeDtypeStruct((_N, _C), jnp.float32),
    mesh=_mesh,
    scratch_types=[
        pltpu.VMEM((_NBUF, _CH, _C), jnp.float32),
        pltpu.VMEM((_RPW,), jnp.int32),
        pltpu.SemaphoreType.DMA((_NBUF,)),
    ],
    compiler_params=pltpu.CompilerParams(needs_layout_passes=False),
)


def kernel(x):
    return _sc_call(x.astype(jnp.int32))
